# baseline (speedup 1.0000x reference)
import functools

import jax
import jax.numpy as jnp
from jax import lax
from jax.experimental import pallas as pl
from jax.experimental.pallas import tpu as pltpu
from jax.experimental.pallas import tpu_sc as plsc

N_NODES = 10000
N_EDGES = 320000
D = 128

NC = 2
NS = 16
CH = 80
NCHUNK = N_EDGES // CH

IB1 = 25
CROWS1 = NCHUNK // NS
NB1 = CROWS1 // IB1
IB = 4
B2 = IB * CH
CROWS2 = NCHUNK // (NC * NS)
NBF = CROWS2 // IB
ZROWS = 624


def _sc_aggregate(x, src2, dst2, zeros2d, zeros1d):
    mesh = plsc.VectorSubcoreMesh(core_axis_name="c", subcore_axis_name="s")

    @functools.partial(
        pl.kernel,
        mesh=mesh,
        out_type=(
            jax.ShapeDtypeStruct((NC, N_NODES, D), jnp.float32),
            jax.ShapeDtypeStruct((N_NODES,), jnp.float32),
            jax.ShapeDtypeStruct((NC, N_NODES), jnp.float32),
        ),
        scratch_types=[
            pltpu.VMEM((2 * IB1 * CH,), jnp.int32),
            pltpu.VMEM((2 * B2,), jnp.int32),
            pltpu.VMEM((2 * B2,), jnp.int32),
            pltpu.VMEM((B2,), jnp.float32),
            pltpu.VMEM((ZROWS,), jnp.float32),
            pltpu.VMEM((CH,), jnp.float32),
            pltpu.VMEM((B2, D), jnp.float32),
            pltpu.VMEM_SHARED((N_NODES, D), jnp.float32),
            pltpu.VMEM_SHARED((N_NODES,), jnp.float32),
            pltpu.VMEM_SHARED((N_NODES,), jnp.float32),
            pltpu.SemaphoreType.DMA,
            pltpu.SemaphoreType.DMA,
            pltpu.SemaphoreType.DMA,
            pltpu.SemaphoreType.DMA,
            pltpu.SemaphoreType.DMA,
            pltpu.SemaphoreType.DMA,
            pltpu.SemaphoreType.DMA,
        ],
    )
    def k(x_hbm, src_hbm, dst_hbm, z2_hbm, z1_hbm,
          out_sum, out_cnt, out_a,
          dstb1, srcb, dstb, wb, recip_v, ones_v, rows_v,
          summed_sh, cnt_sh, a_sh,
          sem_g, sem_c, sem_w, sem_f, sem_1, sem_i, sem_i1):
        c = lax.axis_index("c")
        s = lax.axis_index("s")

        ebase1 = s * (N_EDGES // NS)
        ebase2 = c * (N_EDGES // NC) + s * (N_EDGES // (NC * NS))

        def issue1(i, b):
            return pltpu.async_copy(
                dst_hbm.at[pl.ds(ebase1 + i * IB1 * CH, IB1 * CH)],
                dstb1.at[pl.ds(b * IB1 * CH, IB1 * CH)], sem_i1)

        def issue2(eoff, n, b):
            pltpu.async_copy(src_hbm.at[pl.ds(eoff, n)],
                             srcb.at[pl.ds(b * B2, n)], sem_i)
            pltpu.async_copy(dst_hbm.at[pl.ds(eoff, n)],
                             dstb.at[pl.ds(b * B2, n)], sem_i)

        def wait2(b, n):
            pltpu.make_async_copy(src_hbm.at[pl.ds(0, n)],
                                  srcb.at[pl.ds(b * B2, n)], sem_i).wait()
            pltpu.make_async_copy(dst_hbm.at[pl.ds(0, n)],
                                  dstb.at[pl.ds(b * B2, n)], sem_i).wait()

        issue1(0, 0)
        issue1(1, 1)
        issue2(ebase2, B2, 0)
        issue2(ebase2 + B2, B2, 1)

        pltpu.sync_copy(z2_hbm.at[pl.ds(s * ZROWS, ZROWS)],
                        summed_sh.at[pl.ds(s * ZROWS, ZROWS)])

        @pl.when(s == 0)
        def _():
            pltpu.sync_copy(z2_hbm.at[pl.ds(NS * ZROWS, N_NODES - NS * ZROWS)],
                            summed_sh.at[pl.ds(NS * ZROWS, N_NODES - NS * ZROWS)])
            pltpu.sync_copy(z1_hbm, cnt_sh)

        @pl.when(s == 1)
        def _():
            pltpu.sync_copy(z1_hbm, a_sh)

        for k16 in range(CH // 16):
            ones_v[pl.ds(k16 * 16, 16)] = jnp.ones((16,), jnp.float32)

        plsc.subcore_barrier()

        def process_fblock(b, njc):
            gathers = [
                pltpu.async_copy(x_hbm.at[srcb.at[pl.ds(b * B2 + j * CH, CH)]],
                                 rows_v.at[pl.ds(j * CH, CH)], sem_g)
                for j in range(njc)
            ]
            fscat = []
            for j in range(njc):
                gathers[j].wait()
                fscat.append(
                    pltpu.async_copy(rows_v.at[pl.ds(j * CH, CH)],
                                     summed_sh.at[dstb.at[pl.ds(b * B2 + j * CH, CH)]],
                                     sem_f, add=True))
            for d in fscat:
                d.wait()

        def count_block(fi, b):
            pltpu.make_async_copy(dst_hbm.at[pl.ds(0, IB1 * CH)],
                                  dstb1.at[pl.ds(b * IB1 * CH, IB1 * CH)],
                                  sem_i1).wait()
            scats = [
                pltpu.async_copy(ones_v,
                                 cnt_sh.at[dstb1.at[pl.ds(b * IB1 * CH + j * CH, CH)]],
                                 sem_1, add=True)
                for j in range(IB1)
            ]

            @pl.when(fi + 2 < NB1)
            def _():
                issue1(fi + 2, b)

            for d in scats:
                d.wait()

        @pl.loop(0, NBF - 1, step=2)
        def _(i):
            for b in range(2):
                wait2(b, B2)

                @pl.when(i + b < NB1)
                def _():
                    count_block(i + b, b)

                process_fblock(b, IB)
                nxt = i + b + 2

                @pl.when(nxt < NBF)
                def _():
                    issue2(ebase2 + nxt * B2, B2, b)

                @pl.when(nxt == NBF)
                def _():
                    issue2(ebase2 + NBF * B2, CH, b)

        wait2(0, B2)
        process_fblock(0, IB)
        wait2(1, CH)
        process_fblock(1, 1)

        plsc.subcore_barrier()

        issue2(ebase2, B2, 0)
        issue2(ebase2 + B2, B2, 1)
        pltpu.async_copy(summed_sh.at[pl.ds(s * ZROWS, ZROWS)],
                         out_sum.at[c, pl.ds(s * ZROWS, ZROWS)], sem_f)

        @pl.when(s == 0)
        def _():
            pltpu.async_copy(
                summed_sh.at[pl.ds(NS * ZROWS, N_NODES - NS * ZROWS)],
                out_sum.at[c, pl.ds(NS * ZROWS, N_NODES - NS * ZROWS)], sem_f)

        rbase = s * ZROWS
        pltpu.sync_copy(cnt_sh.at[pl.ds(rbase, ZROWS)], recip_v)
        for k16 in range(ZROWS // 16):
            cv = recip_v[pl.ds(k16 * 16, 16)]
            recip_v[pl.ds(k16 * 16, 16)] = 1.0 / jnp.maximum(cv, 1.0)
        pltpu.sync_copy(recip_v, cnt_sh.at[pl.ds(rbase, ZROWS)])

        @pl.when(s == 0)
        def _():
            pltpu.sync_copy(cnt_sh.at[pl.ds(NS * ZROWS, 16)],
                            recip_v.at[pl.ds(0, 16)])
            cv = recip_v[pl.ds(0, 16)]
            recip_v[pl.ds(0, 16)] = 1.0 / jnp.maximum(cv, 1.0)
            pltpu.sync_copy(recip_v.at[pl.ds(0, 16)],
                            cnt_sh.at[pl.ds(NS * ZROWS, 16)])

        plsc.subcore_barrier()

        @pl.when(jnp.logical_and(s == 1, c == 0))
        def _():
            pltpu.async_copy(cnt_sh, out_cnt, sem_1)

        def process_wblock(b, njc):
            cg = [
                pltpu.async_copy(cnt_sh.at[dstb.at[pl.ds(b * B2 + j * CH, CH)]],
                                 wb.at[pl.ds(j * CH, CH)], sem_c)
                for j in range(njc)
            ]
            ws = []
            for j in range(njc):
                cg[j].wait()
                ws.append(
                    pltpu.async_copy(wb.at[pl.ds(j * CH, CH)],
                                     a_sh.at[srcb.at[pl.ds(b * B2 + j * CH, CH)]],
                                     sem_w, add=True))
            for d in ws:
                d.wait()

        @pl.loop(0, NBF - 1, step=2)
        def _(i):
            for b in range(2):
                wait2(b, B2)
                process_wblock(b, IB)
                nxt = i + b + 2

                @pl.when(nxt < NBF)
                def _():
                    issue2(ebase2 + nxt * B2, B2, b)

                @pl.when(nxt == NBF)
                def _():
                    issue2(ebase2 + NBF * B2, CH, b)

        wait2(0, B2)
        process_wblock(0, IB)
        wait2(1, CH)
        process_wblock(1, 1)

        plsc.subcore_barrier()

        @pl.when(s == 2)
        def _():
            pltpu.sync_copy(a_sh, out_a.at[c])

        pltpu.make_async_copy(summed_sh.at[pl.ds(s * ZROWS, ZROWS)],
                              out_sum.at[c, pl.ds(s * ZROWS, ZROWS)],
                              sem_f).wait()

        @pl.when(s == 0)
        def _():
            pltpu.make_async_copy(
                summed_sh.at[pl.ds(NS * ZROWS, N_NODES - NS * ZROWS)],
                out_sum.at[c, pl.ds(NS * ZROWS, N_NODES - NS * ZROWS)],
                sem_f).wait()

        @pl.when(jnp.logical_and(s == 1, c == 0))
        def _():
            pltpu.make_async_copy(cnt_sh, out_cnt, sem_1).wait()

    return k(x, src2, dst2, zeros2d, zeros1d)


BLK = 1000
NBLK = N_NODES // BLK


def _tc_body(x_ref, sum_ref, cnt_ref, a_ref,
             w1l_ref, w1r_ref, b1_ref, w2l_ref, w2r_ref, b2_ref,
             out_ref, u_acc, v_acc):
    i = pl.program_id(0)

    @pl.when(i == 0)
    def _():
        u_acc[...] = jnp.zeros_like(u_acc)
        v_acc[...] = jnp.zeros_like(v_acc)

    p = sum_ref[0] + sum_ref[1]
    mean = p * cnt_ref[...]
    h = mean @ w1l_ref[...] + b1_ref[...] + x_ref[...] @ w1r_ref[...]
    h = jnp.maximum(h, 0.0)
    a = a_ref[0] + a_ref[1]
    u_acc[...] += jnp.sum(a * h, axis=0, keepdims=True)
    v_acc[...] += jnp.sum(h, axis=0, keepdims=True)

    @pl.when(i == NBLK - 1)
    def _():
        inv_n = 1.0 / N_NODES
        u = u_acc[...] * inv_n
        v = v_acc[...] * inv_n
        out_ref[...] = u @ w2l_ref[...] + b2_ref[...] + v @ w2r_ref[...]


def _tc_fuse(x, summed_p, cnt, a_p, W1_l, W1_r, b1, W2_l, W2_r, b2):
    full = lambda shape: pl.BlockSpec(shape, lambda i: tuple(0 for _ in shape))
    return pl.pallas_call(
        _tc_body,
        grid=(NBLK,),
        in_specs=[
            pl.BlockSpec((BLK, D), lambda i: (i, 0)),
            pl.BlockSpec((NC, BLK, D), lambda i: (0, i, 0)),
            pl.BlockSpec((BLK, 1), lambda i: (i, 0)),
            pl.BlockSpec((NC, BLK, 1), lambda i: (0, i, 0)),
            full((D, D)), full((D, D)), full((1, D)),
            full((D, D)), full((D, D)), full((1, D)),
        ],
        out_specs=pl.BlockSpec((1, D), lambda i: (0, 0)),
        out_shape=jax.ShapeDtypeStruct((1, D), jnp.float32),
        scratch_shapes=[
            pltpu.VMEM((1, D), jnp.float32),
            pltpu.VMEM((1, D), jnp.float32),
        ],
    )(x, summed_p, cnt, a_p, W1_l, W1_r, b1, W2_l, W2_r, b2)


def kernel(x, edge_index, W1_l, W1_r, b1, W2_l, W2_r, b2):
    src2 = edge_index[0].astype(jnp.int32)
    dst2 = edge_index[1].astype(jnp.int32)
    zeros2d = jnp.zeros((N_NODES, D), jnp.float32)
    zeros1d = jnp.zeros((N_NODES,), jnp.float32)

    summed_p, cnt, a_p = _sc_aggregate(x, src2, dst2, zeros2d, zeros1d)

    return _tc_fuse(
        x, summed_p,
        cnt.reshape(N_NODES, 1), a_p.reshape(NC, N_NODES, 1),
        W1_l, W1_r, b1.reshape(1, D), W2_l, W2_r, b2.reshape(1, D),
    )


# speedup vs baseline: 1.0481x; 1.0481x over previous
import functools

import jax
import jax.numpy as jnp
from jax import lax
from jax.experimental import pallas as pl
from jax.experimental.pallas import tpu as pltpu
from jax.experimental.pallas import tpu_sc as plsc

N_NODES = 10000
N_EDGES = 320000
D = 128

NC = 2
NS = 16
CH = 80
NCHUNK = N_EDGES // CH

IB1 = 25
CROWS1 = NCHUNK // NS
NB1 = CROWS1 // IB1
IB = 4
B2 = IB * CH
CROWS2 = NCHUNK // (NC * NS)
NBF = CROWS2 // IB
ZROWS = 624


def _sc_aggregate(x, src2, dst2, zeros2d, zeros1d):
    mesh = plsc.VectorSubcoreMesh(core_axis_name="c", subcore_axis_name="s")

    @functools.partial(
        pl.kernel,
        mesh=mesh,
        out_type=(
            jax.ShapeDtypeStruct((NC, N_NODES, D), jnp.float32),
            jax.ShapeDtypeStruct((N_NODES,), jnp.float32),
            jax.ShapeDtypeStruct((NC, N_NODES), jnp.float32),
        ),
        scratch_types=[
            pltpu.VMEM((2 * IB1 * CH,), jnp.int32),
            pltpu.VMEM((2 * B2,), jnp.int32),
            pltpu.VMEM((2 * B2,), jnp.int32),
            pltpu.VMEM((B2,), jnp.float32),
            pltpu.VMEM((ZROWS,), jnp.float32),
            pltpu.VMEM((CH,), jnp.float32),
            pltpu.VMEM((B2, D), jnp.float32),
            pltpu.VMEM_SHARED((N_NODES, D), jnp.float32),
            pltpu.VMEM_SHARED((N_NODES,), jnp.float32),
            pltpu.VMEM_SHARED((N_NODES,), jnp.float32),
            pltpu.SemaphoreType.DMA,
            pltpu.SemaphoreType.DMA,
            pltpu.SemaphoreType.DMA,
            pltpu.SemaphoreType.DMA,
            pltpu.SemaphoreType.DMA,
            pltpu.SemaphoreType.DMA,
            pltpu.SemaphoreType.DMA,
        ],
    )
    def k(x_hbm, src_hbm, dst_hbm, z2_hbm, z1_hbm,
          out_sum, out_cnt, out_a,
          dstb1, srcb, dstb, wb, recip_v, ones_v, rows_v,
          summed_sh, cnt_sh, a_sh,
          sem_g, sem_c, sem_w, sem_f, sem_1, sem_i, sem_i1):
        c = lax.axis_index("c")
        s = lax.axis_index("s")

        ebase1 = s * (N_EDGES // NS)
        ebase2 = c * (N_EDGES // NC) + s * (N_EDGES // (NC * NS))

        def issue1(i, b):
            return pltpu.async_copy(
                dst_hbm.at[pl.ds(ebase1 + i * IB1 * CH, IB1 * CH)],
                dstb1.at[pl.ds(b * IB1 * CH, IB1 * CH)], sem_i1)

        def issue2(eoff, n, b):
            pltpu.async_copy(src_hbm.at[pl.ds(eoff, n)],
                             srcb.at[pl.ds(b * B2, n)], sem_i)
            pltpu.async_copy(dst_hbm.at[pl.ds(eoff, n)],
                             dstb.at[pl.ds(b * B2, n)], sem_i)

        def wait2(b, n):
            pltpu.make_async_copy(src_hbm.at[pl.ds(0, n)],
                                  srcb.at[pl.ds(b * B2, n)], sem_i).wait()
            pltpu.make_async_copy(dst_hbm.at[pl.ds(0, n)],
                                  dstb.at[pl.ds(b * B2, n)], sem_i).wait()

        issue1(0, 0)
        issue1(1, 1)
        issue2(ebase2, B2, 0)
        issue2(ebase2 + B2, B2, 1)

        pltpu.sync_copy(z2_hbm.at[pl.ds(s * ZROWS, ZROWS)],
                        summed_sh.at[pl.ds(s * ZROWS, ZROWS)])

        @pl.when(s == 0)
        def _():
            pltpu.sync_copy(z2_hbm.at[pl.ds(NS * ZROWS, N_NODES - NS * ZROWS)],
                            summed_sh.at[pl.ds(NS * ZROWS, N_NODES - NS * ZROWS)])
            pltpu.sync_copy(z1_hbm, cnt_sh)

        @pl.when(s == 1)
        def _():
            pltpu.sync_copy(z1_hbm, a_sh)

        for k16 in range(CH // 16):
            ones_v[pl.ds(k16 * 16, 16)] = jnp.ones((16,), jnp.float32)

        plsc.subcore_barrier()

        for i in range(NB1):
            b1 = i % 2
            pltpu.make_async_copy(dst_hbm.at[pl.ds(0, IB1 * CH)],
                                  dstb1.at[pl.ds(b1 * IB1 * CH, IB1 * CH)],
                                  sem_i1).wait()
            scats = [
                pltpu.async_copy(ones_v,
                                 cnt_sh.at[dstb1.at[pl.ds(b1 * IB1 * CH + j * CH, CH)]],
                                 sem_1, add=True)
                for j in range(IB1)
            ]
            if i + 2 < NB1:
                issue1(i + 2, b1)
            for d in scats:
                d.wait()

        plsc.subcore_barrier()

        rbase = s * ZROWS
        pltpu.sync_copy(cnt_sh.at[pl.ds(rbase, ZROWS)], recip_v)
        for k16 in range(ZROWS // 16):
            cv = recip_v[pl.ds(k16 * 16, 16)]
            recip_v[pl.ds(k16 * 16, 16)] = 1.0 / jnp.maximum(cv, 1.0)
        pltpu.sync_copy(recip_v, cnt_sh.at[pl.ds(rbase, ZROWS)])

        @pl.when(s == 0)
        def _():
            pltpu.sync_copy(cnt_sh.at[pl.ds(NS * ZROWS, 16)],
                            recip_v.at[pl.ds(0, 16)])
            cv = recip_v[pl.ds(0, 16)]
            recip_v[pl.ds(0, 16)] = 1.0 / jnp.maximum(cv, 1.0)
            pltpu.sync_copy(recip_v.at[pl.ds(0, 16)],
                            cnt_sh.at[pl.ds(NS * ZROWS, 16)])

        plsc.subcore_barrier()

        @pl.when(jnp.logical_and(s == 1, c == 0))
        def _():
            pltpu.async_copy(cnt_sh, out_cnt, sem_1)

        def process_block(b, njc):
            gathers = [
                pltpu.async_copy(x_hbm.at[srcb.at[pl.ds(b * B2 + j * CH, CH)]],
                                 rows_v.at[pl.ds(j * CH, CH)], sem_g)
                for j in range(njc)
            ]
            cg = [
                pltpu.async_copy(cnt_sh.at[dstb.at[pl.ds(b * B2 + j * CH, CH)]],
                                 wb.at[pl.ds(j * CH, CH)], sem_c)
                for j in range(njc)
            ]
            for d in cg:
                d.wait()
            ws = [
                pltpu.async_copy(wb.at[pl.ds(j * CH, CH)],
                                 a_sh.at[srcb.at[pl.ds(b * B2 + j * CH, CH)]],
                                 sem_w, add=True)
                for j in range(njc)
            ]
            fscat = []
            for j in range(njc):
                gathers[j].wait()
                fscat.append(
                    pltpu.async_copy(rows_v.at[pl.ds(j * CH, CH)],
                                     summed_sh.at[dstb.at[pl.ds(b * B2 + j * CH, CH)]],
                                     sem_f, add=True))
            for d in ws:
                d.wait()
            for d in fscat:
                d.wait()

        @pl.loop(0, NBF - 1, step=2)
        def _(i):
            for b in range(2):
                wait2(b, B2)
                process_block(b, IB)
                nxt = i + b + 2

                @pl.when(nxt < NBF)
                def _():
                    issue2(ebase2 + nxt * B2, B2, b)

                @pl.when(nxt == NBF)
                def _():
                    issue2(ebase2 + NBF * B2, CH, b)

        wait2(0, B2)
        process_block(0, IB)
        wait2(1, CH)
        process_block(1, 1)

        plsc.subcore_barrier()

        pltpu.async_copy(summed_sh.at[pl.ds(s * ZROWS, ZROWS)],
                         out_sum.at[c, pl.ds(s * ZROWS, ZROWS)], sem_f)

        @pl.when(s == 0)
        def _():
            pltpu.async_copy(
                summed_sh.at[pl.ds(NS * ZROWS, N_NODES - NS * ZROWS)],
                out_sum.at[c, pl.ds(NS * ZROWS, N_NODES - NS * ZROWS)], sem_f)

        @pl.when(s == 2)
        def _():
            pltpu.async_copy(a_sh, out_a.at[c], sem_w)

        pltpu.make_async_copy(summed_sh.at[pl.ds(s * ZROWS, ZROWS)],
                              out_sum.at[c, pl.ds(s * ZROWS, ZROWS)],
                              sem_f).wait()

        @pl.when(s == 0)
        def _():
            pltpu.make_async_copy(
                summed_sh.at[pl.ds(NS * ZROWS, N_NODES - NS * ZROWS)],
                out_sum.at[c, pl.ds(NS * ZROWS, N_NODES - NS * ZROWS)],
                sem_f).wait()

        @pl.when(s == 2)
        def _():
            pltpu.make_async_copy(a_sh, out_a.at[c], sem_w).wait()

        @pl.when(jnp.logical_and(s == 1, c == 0))
        def _():
            pltpu.make_async_copy(cnt_sh, out_cnt, sem_1).wait()

    return k(x, src2, dst2, zeros2d, zeros1d)


BLK = 1000
NBLK = N_NODES // BLK


def _tc_body(x_ref, sum_ref, cnt_ref, a_ref,
             w1l_ref, w1r_ref, b1_ref, w2l_ref, w2r_ref, b2_ref,
             out_ref, u_acc, v_acc):
    i = pl.program_id(0)

    @pl.when(i == 0)
    def _():
        u_acc[...] = jnp.zeros_like(u_acc)
        v_acc[...] = jnp.zeros_like(v_acc)

    p = sum_ref[0] + sum_ref[1]
    mean = p * cnt_ref[...]
    h = mean @ w1l_ref[...] + b1_ref[...] + x_ref[...] @ w1r_ref[...]
    h = jnp.maximum(h, 0.0)
    a = a_ref[0] + a_ref[1]
    u_acc[...] += jnp.sum(a * h, axis=0, keepdims=True)
    v_acc[...] += jnp.sum(h, axis=0, keepdims=True)

    @pl.when(i == NBLK - 1)
    def _():
        inv_n = 1.0 / N_NODES
        u = u_acc[...] * inv_n
        v = v_acc[...] * inv_n
        out_ref[...] = u @ w2l_ref[...] + b2_ref[...] + v @ w2r_ref[...]


def _tc_fuse(x, summed_p, cnt, a_p, W1_l, W1_r, b1, W2_l, W2_r, b2):
    full = lambda shape: pl.BlockSpec(shape, lambda i: tuple(0 for _ in shape))
    return pl.pallas_call(
        _tc_body,
        grid=(NBLK,),
        in_specs=[
            pl.BlockSpec((BLK, D), lambda i: (i, 0)),
            pl.BlockSpec((NC, BLK, D), lambda i: (0, i, 0)),
            pl.BlockSpec((BLK, 1), lambda i: (i, 0)),
            pl.BlockSpec((NC, BLK, 1), lambda i: (0, i, 0)),
            full((D, D)), full((D, D)), full((1, D)),
            full((D, D)), full((D, D)), full((1, D)),
        ],
        out_specs=pl.BlockSpec((1, D), lambda i: (0, 0)),
        out_shape=jax.ShapeDtypeStruct((1, D), jnp.float32),
        scratch_shapes=[
            pltpu.VMEM((1, D), jnp.float32),
            pltpu.VMEM((1, D), jnp.float32),
        ],
    )(x, summed_p, cnt, a_p, W1_l, W1_r, b1, W2_l, W2_r, b2)


def kernel(x, edge_index, W1_l, W1_r, b1, W2_l, W2_r, b2):
    src2 = edge_index[0].astype(jnp.int32)
    dst2 = edge_index[1].astype(jnp.int32)
    zeros2d = jnp.zeros((N_NODES, D), jnp.float32)
    zeros1d = jnp.zeros((N_NODES,), jnp.float32)

    summed_p, cnt, a_p = _sc_aggregate(x, src2, dst2, zeros2d, zeros1d)

    return _tc_fuse(
        x, summed_p,
        cnt.reshape(N_NODES, 1), a_p.reshape(NC, N_NODES, 1),
        W1_l, W1_r, b1.reshape(1, D), W2_l, W2_r, b2.reshape(1, D),
    )


# speedup vs baseline: 1.0639x; 1.0151x over previous
import functools

import jax
import jax.numpy as jnp
from jax import lax
from jax.experimental import pallas as pl
from jax.experimental.pallas import tpu as pltpu
from jax.experimental.pallas import tpu_sc as plsc

N_NODES = 10000
N_EDGES = 320000
D = 128

NC = 2
NS = 16
CH = 80
NCHUNK = N_EDGES // CH

IB1 = 25
CROWS1 = NCHUNK // NS
NB1 = CROWS1 // IB1
IB = 2
B2 = IB * CH
CROWS2 = NCHUNK // (NC * NS)
NBF = CROWS2 // IB
ZROWS = 624


def _sc_aggregate(x, src2, dst2, zeros2d, zeros1d):
    mesh = plsc.VectorSubcoreMesh(core_axis_name="c", subcore_axis_name="s")

    @functools.partial(
        pl.kernel,
        mesh=mesh,
        out_type=(
            jax.ShapeDtypeStruct((NC, N_NODES, D), jnp.float32),
            jax.ShapeDtypeStruct((N_NODES,), jnp.float32),
            jax.ShapeDtypeStruct((NC, N_NODES), jnp.float32),
        ),
        scratch_types=[
            pltpu.VMEM((2 * IB1 * CH,), jnp.int32),
            pltpu.VMEM((4 * B2,), jnp.int32),
            pltpu.VMEM((4 * B2,), jnp.int32),
            pltpu.VMEM((2 * B2,), jnp.float32),
            pltpu.VMEM((ZROWS,), jnp.float32),
            pltpu.VMEM((CH,), jnp.float32),
            pltpu.VMEM((2 * B2, D), jnp.float32),
            pltpu.VMEM_SHARED((N_NODES, D), jnp.float32),
            pltpu.VMEM_SHARED((N_NODES,), jnp.float32),
            pltpu.VMEM_SHARED((N_NODES,), jnp.float32),
            pltpu.SemaphoreType.DMA,
            pltpu.SemaphoreType.DMA,
            pltpu.SemaphoreType.DMA,
            pltpu.SemaphoreType.DMA,
            pltpu.SemaphoreType.DMA,
            pltpu.SemaphoreType.DMA,
            pltpu.SemaphoreType.DMA,
        ],
    )
    def k(x_hbm, src_hbm, dst_hbm, z2_hbm, z1_hbm,
          out_sum, out_cnt, out_a,
          dstb1, srcb, dstb, wb, recip_v, ones_v, rows_v,
          summed_sh, cnt_sh, a_sh,
          sem_g, sem_c, sem_w, sem_f, sem_1, sem_i, sem_i1):
        c = lax.axis_index("c")
        s = lax.axis_index("s")

        ebase1 = s * (N_EDGES // NS)
        ebase2 = c * (N_EDGES // NC) + s * (N_EDGES // (NC * NS))

        def issue1(i, b):
            return pltpu.async_copy(
                dst_hbm.at[pl.ds(ebase1 + i * IB1 * CH, IB1 * CH)],
                dstb1.at[pl.ds(b * IB1 * CH, IB1 * CH)], sem_i1)

        def issue2(eoff, n, b):
            pltpu.async_copy(src_hbm.at[pl.ds(eoff, n)],
                             srcb.at[pl.ds(b * B2, n)], sem_i)
            pltpu.async_copy(dst_hbm.at[pl.ds(eoff, n)],
                             dstb.at[pl.ds(b * B2, n)], sem_i)

        def wait2(b, n):
            pltpu.make_async_copy(src_hbm.at[pl.ds(0, n)],
                                  srcb.at[pl.ds(b * B2, n)], sem_i).wait()
            pltpu.make_async_copy(dst_hbm.at[pl.ds(0, n)],
                                  dstb.at[pl.ds(b * B2, n)], sem_i).wait()

        issue1(0, 0)
        issue1(1, 1)
        issue2(ebase2, B2, 0)
        issue2(ebase2 + B2, B2, 1)

        pltpu.sync_copy(z2_hbm.at[pl.ds(s * ZROWS, ZROWS)],
                        summed_sh.at[pl.ds(s * ZROWS, ZROWS)])

        @pl.when(s == 0)
        def _():
            pltpu.sync_copy(z2_hbm.at[pl.ds(NS * ZROWS, N_NODES - NS * ZROWS)],
                            summed_sh.at[pl.ds(NS * ZROWS, N_NODES - NS * ZROWS)])
            pltpu.sync_copy(z1_hbm, cnt_sh)

        @pl.when(s == 1)
        def _():
            pltpu.sync_copy(z1_hbm, a_sh)

        for k16 in range(CH // 16):
            ones_v[pl.ds(k16 * 16, 16)] = jnp.ones((16,), jnp.float32)

        plsc.subcore_barrier()

        for i in range(NB1):
            b1 = i % 2
            pltpu.make_async_copy(dst_hbm.at[pl.ds(0, IB1 * CH)],
                                  dstb1.at[pl.ds(b1 * IB1 * CH, IB1 * CH)],
                                  sem_i1).wait()
            scats = [
                pltpu.async_copy(ones_v,
                                 cnt_sh.at[dstb1.at[pl.ds(b1 * IB1 * CH + j * CH, CH)]],
                                 sem_1, add=True)
                for j in range(IB1)
            ]
            if i + 2 < NB1:
                issue1(i + 2, b1)
            for d in scats:
                d.wait()

        plsc.subcore_barrier()

        rbase = s * ZROWS
        pltpu.sync_copy(cnt_sh.at[pl.ds(rbase, ZROWS)], recip_v)
        for k16 in range(ZROWS // 16):
            cv = recip_v[pl.ds(k16 * 16, 16)]
            recip_v[pl.ds(k16 * 16, 16)] = 1.0 / jnp.maximum(cv, 1.0)
        pltpu.sync_copy(recip_v, cnt_sh.at[pl.ds(rbase, ZROWS)])

        @pl.when(s == 0)
        def _():
            pltpu.sync_copy(cnt_sh.at[pl.ds(NS * ZROWS, 16)],
                            recip_v.at[pl.ds(0, 16)])
            cv = recip_v[pl.ds(0, 16)]
            recip_v[pl.ds(0, 16)] = 1.0 / jnp.maximum(cv, 1.0)
            pltpu.sync_copy(recip_v.at[pl.ds(0, 16)],
                            cnt_sh.at[pl.ds(NS * ZROWS, 16)])

        plsc.subcore_barrier()

        @pl.when(jnp.logical_and(s == 1, c == 0))
        def _():
            pltpu.async_copy(cnt_sh, out_cnt, sem_1)

        def process_block(islot, dslot, njc):
            gathers = [
                pltpu.async_copy(
                    x_hbm.at[srcb.at[pl.ds(islot * B2 + j * CH, CH)]],
                    rows_v.at[pl.ds(dslot * B2 + j * CH, CH)], sem_g)
                for j in range(njc)
            ]
            cg = [
                pltpu.async_copy(
                    cnt_sh.at[dstb.at[pl.ds(islot * B2 + j * CH, CH)]],
                    wb.at[pl.ds(dslot * B2 + j * CH, CH)], sem_c)
                for j in range(njc)
            ]
            for d in cg:
                d.wait()
            for j in range(njc):
                pltpu.async_copy(
                    wb.at[pl.ds(dslot * B2 + j * CH, CH)],
                    a_sh.at[srcb.at[pl.ds(islot * B2 + j * CH, CH)]],
                    sem_w, add=True)
            for j in range(njc):
                gathers[j].wait()
                pltpu.async_copy(
                    rows_v.at[pl.ds(dslot * B2 + j * CH, CH)],
                    summed_sh.at[dstb.at[pl.ds(islot * B2 + j * CH, CH)]],
                    sem_f, add=True)

        def wait_scat(njc):
            for _j in range(njc):
                pltpu.make_async_copy(
                    wb.at[pl.ds(0, CH)],
                    a_sh.at[srcb.at[pl.ds(0, CH)]], sem_w).wait()
            for _j in range(njc):
                pltpu.make_async_copy(
                    rows_v.at[pl.ds(0, CH)],
                    summed_sh.at[dstb.at[pl.ds(0, CH)]], sem_f).wait()

        issue2(ebase2 + 2 * B2, B2, 2)
        wait2(0, B2)
        process_block(0, 0, IB)
        issue2(ebase2 + 3 * B2, B2, 3)
        wait2(1, B2)
        process_block(1, 1, IB)
        wait_scat(IB)
        issue2(ebase2 + 4 * B2, B2, 0)
        wait2(2, B2)
        process_block(2, 0, IB)
        wait_scat(IB)
        issue2(ebase2 + 5 * B2, B2, 1)
        wait2(3, B2)
        process_block(3, 1, IB)

        @pl.loop(4, NBF - 2, step=4)
        def _(i):
            for b in range(4):
                wait_scat(IB)
                issue2(ebase2 + (i + b + 2) * B2, B2, (b + 2) % 4)
                wait2(b, B2)
                process_block(b, b % 2, IB)

        wait_scat(IB)
        issue2(ebase2 + NBF * B2, CH, 2)
        wait2(0, B2)
        process_block(0, 0, IB)
        wait_scat(IB)
        wait2(1, B2)
        process_block(1, 1, IB)
        wait_scat(IB)
        wait2(2, CH)
        process_block(2, 0, 1)
        wait_scat(IB)
        wait_scat(1)

        plsc.subcore_barrier()

        pltpu.async_copy(summed_sh.at[pl.ds(s * ZROWS, ZROWS)],
                         out_sum.at[c, pl.ds(s * ZROWS, ZROWS)], sem_f)

        @pl.when(s == 0)
        def _():
            pltpu.async_copy(
                summed_sh.at[pl.ds(NS * ZROWS, N_NODES - NS * ZROWS)],
                out_sum.at[c, pl.ds(NS * ZROWS, N_NODES - NS * ZROWS)], sem_f)

        @pl.when(s == 2)
        def _():
            pltpu.async_copy(a_sh, out_a.at[c], sem_w)

        pltpu.make_async_copy(summed_sh.at[pl.ds(s * ZROWS, ZROWS)],
                              out_sum.at[c, pl.ds(s * ZROWS, ZROWS)],
                              sem_f).wait()

        @pl.when(s == 0)
        def _():
            pltpu.make_async_copy(
                summed_sh.at[pl.ds(NS * ZROWS, N_NODES - NS * ZROWS)],
                out_sum.at[c, pl.ds(NS * ZROWS, N_NODES - NS * ZROWS)],
                sem_f).wait()

        @pl.when(s == 2)
        def _():
            pltpu.make_async_copy(a_sh, out_a.at[c], sem_w).wait()

        @pl.when(jnp.logical_and(s == 1, c == 0))
        def _():
            pltpu.make_async_copy(cnt_sh, out_cnt, sem_1).wait()

    return k(x, src2, dst2, zeros2d, zeros1d)


BLK = 1000
NBLK = N_NODES // BLK


def _tc_body(x_ref, sum_ref, cnt_ref, a_ref,
             w1l_ref, w1r_ref, b1_ref, w2l_ref, w2r_ref, b2_ref,
             out_ref, u_acc, v_acc):
    i = pl.program_id(0)

    @pl.when(i == 0)
    def _():
        u_acc[...] = jnp.zeros_like(u_acc)
        v_acc[...] = jnp.zeros_like(v_acc)

    p = sum_ref[0] + sum_ref[1]
    mean = p * cnt_ref[...]
    h = mean @ w1l_ref[...] + b1_ref[...] + x_ref[...] @ w1r_ref[...]
    h = jnp.maximum(h, 0.0)
    a = a_ref[0] + a_ref[1]
    u_acc[...] += jnp.sum(a * h, axis=0, keepdims=True)
    v_acc[...] += jnp.sum(h, axis=0, keepdims=True)

    @pl.when(i == NBLK - 1)
    def _():
        inv_n = 1.0 / N_NODES
        u = u_acc[...] * inv_n
        v = v_acc[...] * inv_n
        out_ref[...] = u @ w2l_ref[...] + b2_ref[...] + v @ w2r_ref[...]


def _tc_fuse(x, summed_p, cnt, a_p, W1_l, W1_r, b1, W2_l, W2_r, b2):
    full = lambda shape: pl.BlockSpec(shape, lambda i: tuple(0 for _ in shape))
    return pl.pallas_call(
        _tc_body,
        grid=(NBLK,),
        in_specs=[
            pl.BlockSpec((BLK, D), lambda i: (i, 0)),
            pl.BlockSpec((NC, BLK, D), lambda i: (0, i, 0)),
            pl.BlockSpec((BLK, 1), lambda i: (i, 0)),
            pl.BlockSpec((NC, BLK, 1), lambda i: (0, i, 0)),
            full((D, D)), full((D, D)), full((1, D)),
            full((D, D)), full((D, D)), full((1, D)),
        ],
        out_specs=pl.BlockSpec((1, D), lambda i: (0, 0)),
        out_shape=jax.ShapeDtypeStruct((1, D), jnp.float32),
        scratch_shapes=[
            pltpu.VMEM((1, D), jnp.float32),
            pltpu.VMEM((1, D), jnp.float32),
        ],
    )(x, summed_p, cnt, a_p, W1_l, W1_r, b1, W2_l, W2_r, b2)


def kernel(x, edge_index, W1_l, W1_r, b1, W2_l, W2_r, b2):
    src2 = edge_index[0].astype(jnp.int32)
    dst2 = edge_index[1].astype(jnp.int32)
    zeros2d = jnp.zeros((N_NODES, D), jnp.float32)
    zeros1d = jnp.zeros((N_NODES,), jnp.float32)

    summed_p, cnt, a_p = _sc_aggregate(x, src2, dst2, zeros2d, zeros1d)

    return _tc_fuse(
        x, summed_p,
        cnt.reshape(N_NODES, 1), a_p.reshape(NC, N_NODES, 1),
        W1_l, W1_r, b1.reshape(1, D), W2_l, W2_r, b2.reshape(1, D),
    )


# speedup vs baseline: 1.0739x; 1.0093x over previous
import functools

import jax
import jax.numpy as jnp
from jax import lax
from jax.experimental import pallas as pl
from jax.experimental.pallas import tpu as pltpu
from jax.experimental.pallas import tpu_sc as plsc

N_NODES = 10000
N_EDGES = 320000
D = 128

NC = 2
NS = 16
CH = 80
NCHUNK = N_EDGES // CH

IB1 = 25
CROWS1 = NCHUNK // NS
NB1 = CROWS1 // IB1
IB = 2
B2 = IB * CH
CROWS2 = NCHUNK // (NC * NS)
NBF = CROWS2 // IB
ZROWS = 624


def _sc_aggregate(x, src2, dst2, zeros2d, zeros1d):
    mesh = plsc.VectorSubcoreMesh(core_axis_name="c", subcore_axis_name="s")

    @functools.partial(
        pl.kernel,
        mesh=mesh,
        out_type=(
            jax.ShapeDtypeStruct((NC, N_NODES, D), jnp.float32),
            jax.ShapeDtypeStruct((N_NODES,), jnp.float32),
            jax.ShapeDtypeStruct((NC, N_NODES), jnp.float32),
        ),
        scratch_types=[
            pltpu.VMEM((2 * IB1 * CH,), jnp.int32),
            pltpu.VMEM((4 * B2,), jnp.int32),
            pltpu.VMEM((4 * B2,), jnp.int32),
            pltpu.VMEM((2 * B2,), jnp.float32),
            pltpu.VMEM((ZROWS,), jnp.float32),
            pltpu.VMEM((CH,), jnp.float32),
            pltpu.VMEM((2 * B2, D), jnp.float32),
            pltpu.VMEM_SHARED((N_NODES, D), jnp.float32),
            pltpu.VMEM_SHARED((N_NODES,), jnp.float32),
            pltpu.VMEM_SHARED((N_NODES,), jnp.float32),
            pltpu.SemaphoreType.DMA,
            pltpu.SemaphoreType.DMA,
            pltpu.SemaphoreType.DMA,
            pltpu.SemaphoreType.DMA,
            pltpu.SemaphoreType.DMA,
            pltpu.SemaphoreType.DMA,
            pltpu.SemaphoreType.DMA,
        ],
    )
    def k(x_hbm, src_hbm, dst_hbm, z2_hbm, z1_hbm,
          out_sum, out_cnt, out_a,
          dstb1, srcb, dstb, wb, recip_v, ones_v, rows_v,
          summed_sh, cnt_sh, a_sh,
          sem_g, sem_c, sem_w, sem_f, sem_1, sem_i, sem_i1):
        c = lax.axis_index("c")
        s = lax.axis_index("s")

        ebase1 = s * (N_EDGES // NS)
        ebase2 = c * (N_EDGES // NC) + s * (N_EDGES // (NC * NS))

        def issue1(i, b):
            return pltpu.async_copy(
                dst_hbm.at[pl.ds(ebase1 + i * IB1 * CH, IB1 * CH)],
                dstb1.at[pl.ds(b * IB1 * CH, IB1 * CH)], sem_i1)

        def issue2(eoff, n, b):
            pltpu.async_copy(src_hbm.at[pl.ds(eoff, n)],
                             srcb.at[pl.ds(b * B2, n)], sem_i)
            pltpu.async_copy(dst_hbm.at[pl.ds(eoff, n)],
                             dstb.at[pl.ds(b * B2, n)], sem_i)

        def wait2(b, n):
            pltpu.make_async_copy(src_hbm.at[pl.ds(0, n)],
                                  srcb.at[pl.ds(b * B2, n)], sem_i).wait()
            pltpu.make_async_copy(dst_hbm.at[pl.ds(0, n)],
                                  dstb.at[pl.ds(b * B2, n)], sem_i).wait()

        issue1(0, 0)
        issue1(1, 1)
        issue2(ebase2, B2, 0)
        issue2(ebase2 + B2, B2, 1)

        pltpu.sync_copy(z2_hbm.at[pl.ds(s * ZROWS, ZROWS)],
                        summed_sh.at[pl.ds(s * ZROWS, ZROWS)])

        @pl.when(s == 0)
        def _():
            pltpu.sync_copy(z2_hbm.at[pl.ds(NS * ZROWS, N_NODES - NS * ZROWS)],
                            summed_sh.at[pl.ds(NS * ZROWS, N_NODES - NS * ZROWS)])
            pltpu.sync_copy(z1_hbm, cnt_sh)

        @pl.when(s == 1)
        def _():
            pltpu.sync_copy(z1_hbm, a_sh)

        for k16 in range(CH // 16):
            ones_v[pl.ds(k16 * 16, 16)] = jnp.ones((16,), jnp.float32)

        plsc.subcore_barrier()

        for i in range(NB1):
            b1 = i % 2
            pltpu.make_async_copy(dst_hbm.at[pl.ds(0, IB1 * CH)],
                                  dstb1.at[pl.ds(b1 * IB1 * CH, IB1 * CH)],
                                  sem_i1).wait()
            scats = [
                pltpu.async_copy(ones_v,
                                 cnt_sh.at[dstb1.at[pl.ds(b1 * IB1 * CH + j * CH, CH)]],
                                 sem_1, add=True)
                for j in range(IB1)
            ]
            if i + 2 < NB1:
                issue1(i + 2, b1)
            for d in scats:
                d.wait()

        plsc.subcore_barrier()

        rbase = s * ZROWS
        pltpu.sync_copy(cnt_sh.at[pl.ds(rbase, ZROWS)], recip_v)
        for k16 in range(ZROWS // 16):
            cv = recip_v[pl.ds(k16 * 16, 16)]
            recip_v[pl.ds(k16 * 16, 16)] = 1.0 / jnp.maximum(cv, 1.0)
        pltpu.sync_copy(recip_v, cnt_sh.at[pl.ds(rbase, ZROWS)])

        @pl.when(s == 0)
        def _():
            pltpu.sync_copy(cnt_sh.at[pl.ds(NS * ZROWS, 16)],
                            recip_v.at[pl.ds(0, 16)])
            cv = recip_v[pl.ds(0, 16)]
            recip_v[pl.ds(0, 16)] = 1.0 / jnp.maximum(cv, 1.0)
            pltpu.sync_copy(recip_v.at[pl.ds(0, 16)],
                            cnt_sh.at[pl.ds(NS * ZROWS, 16)])

        plsc.subcore_barrier()

        @pl.when(jnp.logical_and(s == 1, c == 0))
        def _():
            pltpu.async_copy(cnt_sh, out_cnt, sem_1)

        def process_block(islot, dslot, njc):
            gathers = [
                pltpu.async_copy(
                    x_hbm.at[srcb.at[pl.ds(islot * B2 + j * CH, CH)]],
                    rows_v.at[pl.ds(dslot * B2 + j * CH, CH)], sem_g)
                for j in range(njc)
            ]
            cg = [
                pltpu.async_copy(
                    cnt_sh.at[dstb.at[pl.ds(islot * B2 + j * CH, CH)]],
                    wb.at[pl.ds(dslot * B2 + j * CH, CH)], sem_c)
                for j in range(njc)
            ]
            for d in cg:
                d.wait()
            for j in range(njc):
                pltpu.async_copy(
                    wb.at[pl.ds(dslot * B2 + j * CH, CH)],
                    a_sh.at[srcb.at[pl.ds(islot * B2 + j * CH, CH)]],
                    sem_w, add=True)
            for j in range(njc):
                gathers[j].wait()
                pltpu.async_copy(
                    rows_v.at[pl.ds(dslot * B2 + j * CH, CH)],
                    summed_sh.at[dstb.at[pl.ds(islot * B2 + j * CH, CH)]],
                    sem_f, add=True)

        def wait_scat(njc):
            for _j in range(njc):
                pltpu.make_async_copy(
                    wb.at[pl.ds(0, CH)],
                    a_sh.at[srcb.at[pl.ds(0, CH)]], sem_w).wait()
            for _j in range(njc):
                pltpu.make_async_copy(
                    rows_v.at[pl.ds(0, CH)],
                    summed_sh.at[dstb.at[pl.ds(0, CH)]], sem_f).wait()

        issue2(ebase2 + 2 * B2, B2, 2)
        wait2(0, B2)
        process_block(0, 0, IB)
        issue2(ebase2 + 3 * B2, B2, 3)
        wait2(1, B2)
        process_block(1, 1, IB)
        wait_scat(IB)
        issue2(ebase2 + 4 * B2, B2, 0)
        wait2(2, B2)
        process_block(2, 0, IB)
        wait_scat(IB)
        issue2(ebase2 + 5 * B2, B2, 1)
        wait2(3, B2)
        process_block(3, 1, IB)

        @pl.loop(4, NBF - 2, step=4)
        def _(i):
            for b in range(4):
                wait_scat(IB)
                issue2(ebase2 + (i + b + 2) * B2, B2, (b + 2) % 4)
                wait2(b, B2)
                process_block(b, b % 2, IB)

        wait_scat(IB)
        issue2(ebase2 + NBF * B2, CH, 2)
        wait2(0, B2)
        process_block(0, 0, IB)
        wait_scat(IB)
        wait2(1, B2)
        process_block(1, 1, IB)
        wait_scat(IB)
        wait2(2, CH)
        process_block(2, 0, 1)
        wait_scat(IB)
        wait_scat(1)

        plsc.subcore_barrier()

        pltpu.async_copy(summed_sh.at[pl.ds(s * ZROWS, ZROWS)],
                         out_sum.at[c, pl.ds(s * ZROWS, ZROWS)], sem_f)

        @pl.when(s == 0)
        def _():
            pltpu.async_copy(
                summed_sh.at[pl.ds(NS * ZROWS, N_NODES - NS * ZROWS)],
                out_sum.at[c, pl.ds(NS * ZROWS, N_NODES - NS * ZROWS)], sem_f)

        @pl.when(s == 2)
        def _():
            pltpu.async_copy(a_sh, out_a.at[c], sem_w)

        pltpu.make_async_copy(summed_sh.at[pl.ds(s * ZROWS, ZROWS)],
                              out_sum.at[c, pl.ds(s * ZROWS, ZROWS)],
                              sem_f).wait()

        @pl.when(s == 0)
        def _():
            pltpu.make_async_copy(
                summed_sh.at[pl.ds(NS * ZROWS, N_NODES - NS * ZROWS)],
                out_sum.at[c, pl.ds(NS * ZROWS, N_NODES - NS * ZROWS)],
                sem_f).wait()

        @pl.when(s == 2)
        def _():
            pltpu.make_async_copy(a_sh, out_a.at[c], sem_w).wait()

        @pl.when(jnp.logical_and(s == 1, c == 0))
        def _():
            pltpu.make_async_copy(cnt_sh, out_cnt, sem_1).wait()

    return k(x, src2, dst2, zeros2d, zeros1d)


BLK = 2000
NBLK = N_NODES // BLK


def _tc_body(x_ref, sum_ref, cnt_ref, a_ref,
             w1l_ref, w1r_ref, b1_ref, w2l_ref, w2r_ref, b2_ref,
             out_ref, u_acc, v_acc):
    i = pl.program_id(0)

    @pl.when(i == 0)
    def _():
        u_acc[...] = jnp.zeros_like(u_acc)
        v_acc[...] = jnp.zeros_like(v_acc)

    p = sum_ref[0] + sum_ref[1]
    mean = p * cnt_ref[...]
    h = mean @ w1l_ref[...] + b1_ref[...] + x_ref[...] @ w1r_ref[...]
    h = jnp.maximum(h, 0.0)
    a = a_ref[0] + a_ref[1]
    u_acc[...] += jnp.sum(a * h, axis=0, keepdims=True)
    v_acc[...] += jnp.sum(h, axis=0, keepdims=True)

    @pl.when(i == NBLK - 1)
    def _():
        inv_n = 1.0 / N_NODES
        u = u_acc[...] * inv_n
        v = v_acc[...] * inv_n
        out_ref[...] = u @ w2l_ref[...] + b2_ref[...] + v @ w2r_ref[...]


def _tc_fuse(x, summed_p, cnt, a_p, W1_l, W1_r, b1, W2_l, W2_r, b2):
    full = lambda shape: pl.BlockSpec(shape, lambda i: tuple(0 for _ in shape))
    return pl.pallas_call(
        _tc_body,
        grid=(NBLK,),
        in_specs=[
            pl.BlockSpec((BLK, D), lambda i: (i, 0)),
            pl.BlockSpec((NC, BLK, D), lambda i: (0, i, 0)),
            pl.BlockSpec((BLK, 1), lambda i: (i, 0)),
            pl.BlockSpec((NC, BLK, 1), lambda i: (0, i, 0)),
            full((D, D)), full((D, D)), full((1, D)),
            full((D, D)), full((D, D)), full((1, D)),
        ],
        out_specs=pl.BlockSpec((1, D), lambda i: (0, 0)),
        out_shape=jax.ShapeDtypeStruct((1, D), jnp.float32),
        scratch_shapes=[
            pltpu.VMEM((1, D), jnp.float32),
            pltpu.VMEM((1, D), jnp.float32),
        ],
    )(x, summed_p, cnt, a_p, W1_l, W1_r, b1, W2_l, W2_r, b2)


def kernel(x, edge_index, W1_l, W1_r, b1, W2_l, W2_r, b2):
    src2 = edge_index[0].astype(jnp.int32)
    dst2 = edge_index[1].astype(jnp.int32)
    zeros2d = jnp.zeros((N_NODES, D), jnp.float32)
    zeros1d = jnp.zeros((N_NODES,), jnp.float32)

    summed_p, cnt, a_p = _sc_aggregate(x, src2, dst2, zeros2d, zeros1d)

    return _tc_fuse(
        x, summed_p,
        cnt.reshape(N_NODES, 1), a_p.reshape(NC, N_NODES, 1),
        W1_l, W1_r, b1.reshape(1, D), W2_l, W2_r, b2.reshape(1, D),
    )
